# baseline (speedup 1.0000x reference)
import functools

import jax
import jax.numpy as jnp
from jax import lax
from jax.experimental import pallas as pl
from jax.experimental.pallas import tpu as pltpu
from jax.experimental.pallas import tpu_sc as plsc

_D = 64
_L = 128
_NK = 5
_NBUF = 2


@functools.lru_cache(maxsize=None)
def _build(num_rows: int, sc_rows: int):
    info = plsc.get_sparse_core_info()
    nw = info.num_cores * info.num_subcores
    rows_per_w = sc_rows // nw
    n_chunks = rows_per_w // _NK
    assert sc_rows % nw == 0 and rows_per_w % _NK == 0 and n_chunks % 2 == 0

    mesh = plsc.VectorSubcoreMesh(core_axis_name="c", subcore_axis_name="s")

    @functools.partial(
        pl.kernel,
        mesh=mesh,
        compiler_params=pltpu.CompilerParams(use_tc_tiling_on_sc=False),
        out_type=jax.ShapeDtypeStruct((num_rows, _L, _D), jnp.float32),
        scratch_types=[
            pltpu.VMEM((_NBUF, _NK, _L), jnp.int32),
            pltpu.VMEM((_NBUF, _NK, _L, _D), jnp.float32),
            pltpu.SemaphoreType.DMA,
            pltpu.SemaphoreType.DMA,
            pltpu.SemaphoreType.DMA,
            pltpu.SemaphoreType.DMA,
            pltpu.SemaphoreType.DMA,
            pltpu.SemaphoreType.DMA,
            pltpu.VMEM_SHARED((2048, _D), jnp.float32),
        ],
    )
    def gather_kernel(x_hbm, table_hbm, out_hbm, idx_v, rows_v,
                      g0, g1, s0, s1, i0, i1, table_sp):
        cid = lax.axis_index("c")
        sid = lax.axis_index("s")
        wid = sid * info.num_cores + cid
        base = wid * rows_per_w
        gsems = (g0, g1)
        ssems = (s0, s1)
        isems = (i0, i1)

        def load_idx(b, ci):
            row0 = base + ci * _NK
            pltpu.async_copy(x_hbm.at[pl.ds(row0, _NK)], idx_v.at[b],
                             isems[b])

        def wait_idx(b, ci):
            row0 = base + ci * _NK
            pltpu.make_async_copy(x_hbm.at[pl.ds(row0, _NK)], idx_v.at[b],
                                  isems[b]).wait()

        def fire(b):
            for j in range(_NK):
                pltpu.async_copy(table_sp.at[idx_v.at[b].at[j]],
                                 rows_v.at[b].at[j], gsems[b])

        def drain_gathers(b, ci):
            row0 = base + ci * _NK
            pltpu.make_async_copy(out_hbm.at[pl.ds(row0, _NK)],
                                  rows_v.at[b], gsems[b]).wait()

        def store(b, ci):
            row0 = base + ci * _NK
            pltpu.async_copy(rows_v.at[b], out_hbm.at[pl.ds(row0, _NK)],
                             ssems[b])

        def drain_store(b, ci):
            row0 = base + ci * _NK
            pltpu.make_async_copy(rows_v.at[b],
                                  out_hbm.at[pl.ds(row0, _NK)],
                                  ssems[b]).wait()

        @pl.when(sid == 0)
        def _():
            pltpu.sync_copy(table_hbm, table_sp)
        plsc.subcore_barrier()

        for b in range(_NBUF):
            load_idx(b, b)
        for b in range(_NBUF):
            wait_idx(b, b)
            fire(b)

        def body(i, carry):
            for b in range(_NBUF):
                ci = i * _NBUF + b
                drain_gathers(b, ci)
                store(b, ci)
                load_idx(b, ci + _NBUF)
                drain_store(b, ci)
                wait_idx(b, ci + _NBUF)
                fire(b)
            return carry

        lax.fori_loop(0, (n_chunks - _NBUF) // _NBUF, body, 0)

        for b in range(_NBUF):
            ci = n_chunks - _NBUF + b
            drain_gathers(b, ci)
            store(b, ci)
        for b in range(_NBUF):
            drain_store(b, n_chunks - _NBUF + b)

    return gather_kernel


_TCB = 512
_V = 2048


def _tc_lookup(x_flat, table_bf):
    n_idx = x_flat.shape[0]

    def body(idx_ref, tab_ref, o_ref):
        k = lax.broadcasted_iota(jnp.int32, (_TCB, _V), 1)
        p = (idx_ref[...] == k).astype(jnp.bfloat16)
        o_ref[...] = jnp.dot(p, tab_ref[...],
                             preferred_element_type=jnp.float32)

    return pl.pallas_call(
        body,
        grid=(n_idx // _TCB,),
        in_specs=[
            pl.BlockSpec((_TCB, 1), lambda i: (i, 0)),
            pl.BlockSpec((_V, _D), lambda i: (0, 0)),
        ],
        out_specs=pl.BlockSpec((_TCB, _D), lambda i: (i, 0)),
        out_shape=jax.ShapeDtypeStruct((n_idx, _D), jnp.float32),
    )(x_flat.reshape(n_idx, 1), table_bf)


def kernel(x, table):
    s0, s1 = x.shape
    total = s0 * s1
    num_rows = total // _L
    sc_rows = (num_rows * 3 // 4) // 320 * 320
    x_rows = x.reshape(num_rows, _L).astype(jnp.int32)
    out = _build(num_rows, sc_rows)(x_rows[:sc_rows], table)
    x_tc = x_rows[sc_rows:].reshape(-1)
    tc = _tc_lookup(x_tc, table.astype(jnp.bfloat16))
    out = lax.dynamic_update_slice(
        out, tc.reshape(num_rows - sc_rows, _L, _D), (sc_rows, 0, 0))
    return out.reshape(s0, s1, _D)


# speedup vs baseline: 1.1330x; 1.1330x over previous
import functools

import jax
import jax.numpy as jnp
from jax import lax
from jax.experimental import pallas as pl
from jax.experimental.pallas import tpu as pltpu
from jax.experimental.pallas import tpu_sc as plsc

_D = 64
_L = 128
_NK = 5
_NBUF = 2


@functools.lru_cache(maxsize=None)
def _build(num_rows: int, sc_rows: int):
    info = plsc.get_sparse_core_info()
    nw = info.num_cores * info.num_subcores
    rows_per_w = sc_rows // nw
    n_chunks = rows_per_w // _NK
    assert sc_rows % nw == 0 and rows_per_w % _NK == 0 and n_chunks % 2 == 0

    mesh = plsc.VectorSubcoreMesh(core_axis_name="c", subcore_axis_name="s")

    @functools.partial(
        pl.kernel,
        mesh=mesh,
        compiler_params=pltpu.CompilerParams(use_tc_tiling_on_sc=False),
        out_type=jax.ShapeDtypeStruct((num_rows, _L, _D), jnp.float32),
        scratch_types=[
            pltpu.VMEM((_NBUF, _NK, _L), jnp.int32),
            pltpu.VMEM((_NBUF, _NK, _L, _D), jnp.float32),
            pltpu.SemaphoreType.DMA,
            pltpu.SemaphoreType.DMA,
            pltpu.SemaphoreType.DMA,
            pltpu.SemaphoreType.DMA,
            pltpu.SemaphoreType.DMA,
            pltpu.SemaphoreType.DMA,
            pltpu.VMEM_SHARED((2048, _D), jnp.float32),
        ],
    )
    def gather_kernel(x_hbm, table_hbm, out_hbm, idx_v, rows_v,
                      g0, g1, s0, s1, i0, i1, table_sp):
        cid = lax.axis_index("c")
        sid = lax.axis_index("s")
        wid = sid * info.num_cores + cid
        base = wid * rows_per_w
        gsems = (g0, g1)
        ssems = (s0, s1)
        isems = (i0, i1)

        def load_idx(b, ci):
            row0 = base + ci * _NK
            pltpu.async_copy(x_hbm.at[pl.ds(row0, _NK)], idx_v.at[b],
                             isems[b])

        def wait_idx(b, ci):
            row0 = base + ci * _NK
            pltpu.make_async_copy(x_hbm.at[pl.ds(row0, _NK)], idx_v.at[b],
                                  isems[b]).wait()

        def fire(b):
            for j in range(_NK):
                pltpu.async_copy(table_sp.at[idx_v.at[b].at[j]],
                                 rows_v.at[b].at[j], gsems[b])

        def drain_gathers(b, ci):
            row0 = base + ci * _NK
            pltpu.make_async_copy(out_hbm.at[pl.ds(row0, _NK)],
                                  rows_v.at[b], gsems[b]).wait()

        def store(b, ci):
            row0 = base + ci * _NK
            pltpu.async_copy(rows_v.at[b], out_hbm.at[pl.ds(row0, _NK)],
                             ssems[b])

        def drain_store(b, ci):
            row0 = base + ci * _NK
            pltpu.make_async_copy(rows_v.at[b],
                                  out_hbm.at[pl.ds(row0, _NK)],
                                  ssems[b]).wait()

        @pl.when(sid == 0)
        def _():
            pltpu.sync_copy(table_hbm, table_sp)
        plsc.subcore_barrier()

        for b in range(_NBUF):
            load_idx(b, b)
        for b in range(_NBUF):
            wait_idx(b, b)
            fire(b)

        def body(i, carry):
            for b in range(_NBUF):
                ci = i * _NBUF + b
                drain_gathers(b, ci)
                store(b, ci)
                load_idx(b, ci + _NBUF)
                drain_store(b, ci)
                wait_idx(b, ci + _NBUF)
                fire(b)
            return carry

        lax.fori_loop(0, (n_chunks - _NBUF) // _NBUF, body, 0)

        for b in range(_NBUF):
            ci = n_chunks - _NBUF + b
            drain_gathers(b, ci)
            store(b, ci)
        for b in range(_NBUF):
            drain_store(b, n_chunks - _NBUF + b)

    return gather_kernel


_TCB = 512
_V = 2048


def _tc_lookup(x_flat, table_bf):
    n_idx = x_flat.shape[0]

    def body(idx_ref, tab_ref, o_ref):
        k = lax.broadcasted_iota(jnp.int32, (_V, _TCB), 0)
        p = (idx_ref[0] == k).astype(jnp.bfloat16)
        o_ref[...] = lax.dot_general(
            p, tab_ref[...], (((0,), (0,)), ((), ())),
            preferred_element_type=jnp.float32)

    return pl.pallas_call(
        body,
        grid=(n_idx // _TCB,),
        in_specs=[
            pl.BlockSpec((1, 1, _TCB), lambda i: (i, 0, 0)),
            pl.BlockSpec((_V, _D), lambda i: (0, 0)),
        ],
        out_specs=pl.BlockSpec((_TCB, _D), lambda i: (i, 0)),
        out_shape=jax.ShapeDtypeStruct((n_idx, _D), jnp.float32),
    )(x_flat.reshape(n_idx // _TCB, 1, _TCB), table_bf)


def kernel(x, table):
    s0, s1 = x.shape
    total = s0 * s1
    num_rows = total // _L
    sc_rows = (num_rows * 3 // 4) // 320 * 320
    x_rows = x.reshape(num_rows, _L).astype(jnp.int32)
    out = _build(num_rows, sc_rows)(x_rows[:sc_rows], table)
    x_tc = x_rows[sc_rows:].reshape(-1)
    tc = _tc_lookup(x_tc, table.astype(jnp.bfloat16))
    out = lax.dynamic_update_slice(
        out, tc.reshape(num_rows - sc_rows, _L, _D), (sc_rows, 0, 0))
    return out.reshape(s0, s1, _D)


# speedup vs baseline: 1.2109x; 1.0687x over previous
import functools

import jax
import jax.numpy as jnp
from jax import lax
from jax.experimental import pallas as pl
from jax.experimental.pallas import tpu as pltpu
from jax.experimental.pallas import tpu_sc as plsc

_D = 64
_L = 128
_NK = 5
_NBUF = 2


@functools.lru_cache(maxsize=None)
def _build(num_rows: int, sc_rows: int):
    info = plsc.get_sparse_core_info()
    nw = info.num_cores * info.num_subcores
    rows_per_w = sc_rows // nw
    n_chunks = rows_per_w // _NK
    assert sc_rows % nw == 0 and rows_per_w % _NK == 0 and n_chunks % 2 == 0

    mesh = plsc.VectorSubcoreMesh(core_axis_name="c", subcore_axis_name="s")

    @functools.partial(
        pl.kernel,
        mesh=mesh,
        compiler_params=pltpu.CompilerParams(use_tc_tiling_on_sc=False),
        out_type=jax.ShapeDtypeStruct((num_rows, _L, _D), jnp.float32),
        scratch_types=[
            pltpu.VMEM((_NBUF, _NK, _L), jnp.int32),
            pltpu.VMEM((_NBUF, _NK, _L, _D), jnp.float32),
            pltpu.SemaphoreType.DMA,
            pltpu.SemaphoreType.DMA,
            pltpu.SemaphoreType.DMA,
            pltpu.SemaphoreType.DMA,
            pltpu.SemaphoreType.DMA,
            pltpu.SemaphoreType.DMA,
            pltpu.VMEM_SHARED((2048, _D), jnp.float32),
        ],
    )
    def gather_kernel(x_hbm, table_hbm, out_hbm, idx_v, rows_v,
                      g0, g1, s0, s1, i0, i1, table_sp):
        cid = lax.axis_index("c")
        sid = lax.axis_index("s")
        wid = sid * info.num_cores + cid
        base = wid * rows_per_w
        gsems = (g0, g1)
        ssems = (s0, s1)
        isems = (i0, i1)

        def load_idx(b, ci):
            row0 = base + ci * _NK
            pltpu.async_copy(x_hbm.at[pl.ds(row0, _NK)], idx_v.at[b],
                             isems[b])

        def wait_idx(b, ci):
            row0 = base + ci * _NK
            pltpu.make_async_copy(x_hbm.at[pl.ds(row0, _NK)], idx_v.at[b],
                                  isems[b]).wait()

        def fire(b):
            for j in range(_NK):
                pltpu.async_copy(table_sp.at[idx_v.at[b].at[j]],
                                 rows_v.at[b].at[j], gsems[b])

        def drain_gathers(b, ci):
            row0 = base + ci * _NK
            pltpu.make_async_copy(out_hbm.at[pl.ds(row0, _NK)],
                                  rows_v.at[b], gsems[b]).wait()

        def store(b, ci):
            row0 = base + ci * _NK
            pltpu.async_copy(rows_v.at[b], out_hbm.at[pl.ds(row0, _NK)],
                             ssems[b])

        def drain_store(b, ci):
            row0 = base + ci * _NK
            pltpu.make_async_copy(rows_v.at[b],
                                  out_hbm.at[pl.ds(row0, _NK)],
                                  ssems[b]).wait()

        @pl.when(sid == 0)
        def _():
            pltpu.sync_copy(table_hbm, table_sp)
        plsc.subcore_barrier()

        for b in range(_NBUF):
            load_idx(b, b)
        for b in range(_NBUF):
            wait_idx(b, b)
            fire(b)

        def body(i, carry):
            for b in range(_NBUF):
                ci = i * _NBUF + b
                drain_gathers(b, ci)
                store(b, ci)
                load_idx(b, ci + _NBUF)
                drain_store(b, ci)
                wait_idx(b, ci + _NBUF)
                fire(b)
            return carry

        lax.fori_loop(0, (n_chunks - _NBUF) // _NBUF, body, 0)

        for b in range(_NBUF):
            ci = n_chunks - _NBUF + b
            drain_gathers(b, ci)
            store(b, ci)
        for b in range(_NBUF):
            drain_store(b, n_chunks - _NBUF + b)

    return gather_kernel


_TCB = 1024
_V = 2048


def _tc_lookup(x_flat, table_bf):
    n_idx = x_flat.shape[0]

    def body(idx_ref, tab_ref, o_ref):
        k = lax.broadcasted_iota(jnp.int32, (_V, _TCB), 0)
        p = (idx_ref[0] == k).astype(jnp.bfloat16)
        o_ref[...] = lax.dot_general(
            p, tab_ref[...], (((0,), (0,)), ((), ())),
            preferred_element_type=jnp.float32)

    return pl.pallas_call(
        body,
        grid=(n_idx // _TCB,),
        in_specs=[
            pl.BlockSpec((1, 1, _TCB), lambda i: (i, 0, 0)),
            pl.BlockSpec((_V, _D), lambda i: (0, 0)),
        ],
        out_specs=pl.BlockSpec((_TCB, _D), lambda i: (i, 0)),
        out_shape=jax.ShapeDtypeStruct((n_idx, _D), jnp.float32),
    )(x_flat.reshape(n_idx // _TCB, 1, _TCB), table_bf)


def kernel(x, table):
    s0, s1 = x.shape
    total = s0 * s1
    num_rows = total // _L
    sc_rows = (num_rows * 3 // 4) // 320 * 320
    x_rows = x.reshape(num_rows, _L).astype(jnp.int32)
    out = _build(num_rows, sc_rows)(x_rows[:sc_rows], table)
    x_tc = x_rows[sc_rows:].reshape(-1)
    tc = _tc_lookup(x_tc, table.astype(jnp.bfloat16))
    out = lax.dynamic_update_slice(
        out, tc.reshape(num_rows - sc_rows, _L, _D), (sc_rows, 0, 0))
    return out.reshape(s0, s1, _D)


# speedup vs baseline: 1.3420x; 1.1082x over previous
import functools

import jax
import jax.numpy as jnp
from jax import lax
from jax.experimental import pallas as pl
from jax.experimental.pallas import tpu as pltpu
from jax.experimental.pallas import tpu_sc as plsc

_D = 64
_L = 128
_NK = 5
_NBUF = 2


@functools.lru_cache(maxsize=None)
def _build(num_rows: int, sc_rows: int):
    info = plsc.get_sparse_core_info()
    nw = info.num_cores * info.num_subcores
    rows_per_w = sc_rows // nw
    n_chunks = rows_per_w // _NK
    assert sc_rows % nw == 0 and rows_per_w % _NK == 0 and n_chunks % 2 == 0

    mesh = plsc.VectorSubcoreMesh(core_axis_name="c", subcore_axis_name="s")

    @functools.partial(
        pl.kernel,
        mesh=mesh,
        compiler_params=pltpu.CompilerParams(use_tc_tiling_on_sc=False),
        out_type=jax.ShapeDtypeStruct((num_rows, _L, _D), jnp.float32),
        scratch_types=[
            pltpu.VMEM((_NBUF, _NK, _L), jnp.int32),
            pltpu.VMEM((_NBUF, _NK, _L, _D), jnp.float32),
            pltpu.SemaphoreType.DMA,
            pltpu.SemaphoreType.DMA,
            pltpu.SemaphoreType.DMA,
            pltpu.SemaphoreType.DMA,
            pltpu.SemaphoreType.DMA,
            pltpu.SemaphoreType.DMA,
            pltpu.VMEM_SHARED((2048, _D), jnp.float32),
        ],
    )
    def gather_kernel(x_hbm, table_hbm, out_hbm, idx_v, rows_v,
                      g0, g1, s0, s1, i0, i1, table_sp):
        cid = lax.axis_index("c")
        sid = lax.axis_index("s")
        wid = sid * info.num_cores + cid
        base = wid * rows_per_w
        gsems = (g0, g1)
        ssems = (s0, s1)
        isems = (i0, i1)

        def load_idx(b, ci):
            row0 = base + ci * _NK
            pltpu.async_copy(x_hbm.at[pl.ds(row0, _NK)], idx_v.at[b],
                             isems[b])

        def wait_idx(b, ci):
            row0 = base + ci * _NK
            pltpu.make_async_copy(x_hbm.at[pl.ds(row0, _NK)], idx_v.at[b],
                                  isems[b]).wait()

        def fire(b):
            for j in range(_NK):
                pltpu.async_copy(table_sp.at[idx_v.at[b].at[j]],
                                 rows_v.at[b].at[j], gsems[b])

        def drain_gathers(b, ci):
            row0 = base + ci * _NK
            pltpu.make_async_copy(out_hbm.at[pl.ds(row0, _NK)],
                                  rows_v.at[b], gsems[b]).wait()

        def store(b, ci):
            row0 = base + ci * _NK
            pltpu.async_copy(rows_v.at[b], out_hbm.at[pl.ds(row0, _NK)],
                             ssems[b])

        def drain_store(b, ci):
            row0 = base + ci * _NK
            pltpu.make_async_copy(rows_v.at[b],
                                  out_hbm.at[pl.ds(row0, _NK)],
                                  ssems[b]).wait()

        @pl.when(sid == 0)
        def _():
            pltpu.sync_copy(table_hbm, table_sp)
        plsc.subcore_barrier()

        for b in range(_NBUF):
            load_idx(b, b)
        for b in range(_NBUF):
            wait_idx(b, b)
            fire(b)

        def body(i, carry):
            for b in range(_NBUF):
                ci = i * _NBUF + b
                drain_gathers(b, ci)
                store(b, ci)
                load_idx(b, ci + _NBUF)
                drain_store(b, ci)
                wait_idx(b, ci + _NBUF)
                fire(b)
            return carry

        lax.fori_loop(0, (n_chunks - _NBUF) // _NBUF, body, 0)

        for b in range(_NBUF):
            ci = n_chunks - _NBUF + b
            drain_gathers(b, ci)
            store(b, ci)
        for b in range(_NBUF):
            drain_store(b, n_chunks - _NBUF + b)

    return gather_kernel


_TCB = 1024
_V = 2048


def _tc_lookup(x_flat, table_bf):
    n_idx = x_flat.shape[0]

    def body(idx_ref, tab_ref, o_ref):
        k = lax.broadcasted_iota(jnp.int32, (_V, _TCB), 0)
        p = (idx_ref[0] == k).astype(jnp.bfloat16)
        o_ref[...] = lax.dot_general(
            p, tab_ref[...], (((0,), (0,)), ((), ())),
            preferred_element_type=jnp.float32)

    return pl.pallas_call(
        body,
        grid=(n_idx // _TCB,),
        in_specs=[
            pl.BlockSpec((1, 1, _TCB), lambda i: (i, 0, 0)),
            pl.BlockSpec((_V, _D), lambda i: (0, 0)),
        ],
        out_specs=pl.BlockSpec((_TCB, _D), lambda i: (i, 0)),
        out_shape=jax.ShapeDtypeStruct((n_idx, _D), jnp.float32),
    )(x_flat.reshape(n_idx // _TCB, 1, _TCB), table_bf)


def kernel(x, table):
    s0, s1 = x.shape
    total = s0 * s1
    num_rows = total // _L
    sc_rows = (num_rows * 13 // 16) // 320 * 320
    x_rows = x.reshape(num_rows, _L).astype(jnp.int32)
    out = _build(num_rows, sc_rows)(x_rows[:sc_rows], table)
    x_tc = x_rows[sc_rows:].reshape(-1)
    tc = _tc_lookup(x_tc, table.astype(jnp.bfloat16))
    out = lax.dynamic_update_slice(
        out, tc.reshape(num_rows - sc_rows, _L, _D), (sc_rows, 0, 0))
    return out.reshape(s0, s1, _D)


# speedup vs baseline: 1.7600x; 1.3115x over previous
import functools

import jax
import jax.numpy as jnp
from jax import lax
from jax.experimental import pallas as pl
from jax.experimental.pallas import tpu as pltpu
from jax.experimental.pallas import tpu_sc as plsc

_D = 64
_L = 128
_NK = 5
_NBUF = 2


@functools.lru_cache(maxsize=None)
def _build(num_rows: int):
    info = plsc.get_sparse_core_info()
    nw = info.num_cores * info.num_subcores
    rows_per_w = num_rows // nw
    n_chunks = rows_per_w // _NK
    assert num_rows % nw == 0 and rows_per_w % _NK == 0 and n_chunks % 2 == 0

    mesh = plsc.VectorSubcoreMesh(core_axis_name="c", subcore_axis_name="s")

    @functools.partial(
        pl.kernel,
        mesh=mesh,
        compiler_params=pltpu.CompilerParams(use_tc_tiling_on_sc=False),
        out_type=jax.ShapeDtypeStruct((num_rows, _L, _D), jnp.float32),
        scratch_types=[
            pltpu.VMEM((_NBUF, _NK, _L), jnp.int32),
            pltpu.VMEM((_NBUF, _NK, _L, _D), jnp.float32),
            pltpu.SemaphoreType.DMA,
            pltpu.SemaphoreType.DMA,
            pltpu.SemaphoreType.DMA,
            pltpu.SemaphoreType.DMA,
            pltpu.SemaphoreType.DMA,
            pltpu.SemaphoreType.DMA,
            pltpu.VMEM_SHARED((2048, _D), jnp.float32),
        ],
    )
    def gather_kernel(x_hbm, table_hbm, out_hbm, idx_v, rows_v,
                      g0, g1, s0, s1, i0, i1, table_sp):
        cid = lax.axis_index("c")
        sid = lax.axis_index("s")
        wid = sid * info.num_cores + cid
        base = wid * rows_per_w
        gsems = (g0, g1)
        ssems = (s0, s1)
        isems = (i0, i1)

        def load_idx(b, ci):
            row0 = base + ci * _NK
            pltpu.async_copy(x_hbm.at[pl.ds(row0, _NK)], idx_v.at[b],
                             isems[b])

        def wait_idx(b, ci):
            row0 = base + ci * _NK
            pltpu.make_async_copy(x_hbm.at[pl.ds(row0, _NK)], idx_v.at[b],
                                  isems[b]).wait()

        def fire(b):
            for j in range(_NK):
                pltpu.async_copy(table_sp.at[idx_v.at[b].at[j]],
                                 rows_v.at[b].at[j], gsems[b])

        def drain_gathers(b, ci):
            row0 = base + ci * _NK
            pltpu.make_async_copy(out_hbm.at[pl.ds(row0, _NK)],
                                  rows_v.at[b], gsems[b]).wait()

        def store(b, ci):
            row0 = base + ci * _NK
            pltpu.async_copy(rows_v.at[b], out_hbm.at[pl.ds(row0, _NK)],
                             ssems[b])

        def drain_store(b, ci):
            row0 = base + ci * _NK
            pltpu.make_async_copy(rows_v.at[b],
                                  out_hbm.at[pl.ds(row0, _NK)],
                                  ssems[b]).wait()

        @pl.when(sid == 0)
        def _():
            pltpu.sync_copy(table_hbm, table_sp)
        plsc.subcore_barrier()

        for b in range(_NBUF):
            load_idx(b, b)
        for b in range(_NBUF):
            wait_idx(b, b)
            fire(b)

        def body(i, carry):
            for b in range(_NBUF):
                ci = i * _NBUF + b
                drain_gathers(b, ci)
                store(b, ci)
                load_idx(b, ci + _NBUF)
                drain_store(b, ci)
                wait_idx(b, ci + _NBUF)
                fire(b)
            return carry

        lax.fori_loop(0, (n_chunks - _NBUF) // _NBUF, body, 0)

        for b in range(_NBUF):
            ci = n_chunks - _NBUF + b
            drain_gathers(b, ci)
            store(b, ci)
        for b in range(_NBUF):
            drain_store(b, n_chunks - _NBUF + b)

    return gather_kernel


def kernel(x, table):
    s0, s1 = x.shape
    total = s0 * s1
    num_rows = total // _L
    x_rows = x.reshape(num_rows, _L).astype(jnp.int32)
    out = _build(num_rows)(x_rows, table)
    return out.reshape(s0, s1, _D)
